# Initial kernel scaffold; baseline (speedup 1.0000x reference)
import jax
import jax.numpy as jnp
from jax.experimental import pallas as pl


def kernel(x, edge_index, W1l, b1, W1r, W2l, b2, W2r, Wc, bc):
    raise NotImplementedError("write your pallas kernel here")


# speedup vs baseline: 2.1201x; 2.1201x over previous
import jax
import jax.numpy as jnp
from jax import lax
from jax.experimental import pallas as pl
from jax.experimental.pallas import tpu as pltpu
from jax.experimental.pallas import tpu_sc as plsc

_NC = 2
_NS = 16
_B = 128
_LW = 128


def _make_sc_agg(num_chunks: int, n_nodes: int, n_acc: int, nb_per_tile: int,
                 with_deg: bool):
    rpt = n_acc // _NS
    assert rpt % _B == 0

    mesh = plsc.VectorSubcoreMesh(core_axis_name="c", subcore_axis_name="s")

    out_type = [jax.ShapeDtypeStruct((_NC, num_chunks, n_acc, _LW), jnp.float32)]
    if with_deg:
        out_type.append(jax.ShapeDtypeStruct((_NC * _NS, n_acc), jnp.float32))

    scratch = [
        pltpu.VMEM_SHARED((n_acc, _LW), jnp.float32),
        pltpu.VMEM((nb_per_tile, _B), jnp.int32),
        pltpu.VMEM((nb_per_tile, _B), jnp.int32),
        pltpu.VMEM((_B, _LW), jnp.float32),
    ]
    if with_deg:
        scratch.append(pltpu.VMEM((n_acc,), jnp.float32))

    def body(*refs):
        tables = refs[:num_chunks]
        src_hbm, dst_hbm = refs[num_chunks], refs[num_chunks + 1]
        k = num_chunks + 2
        if with_deg:
            part_out, deg_out = refs[k], refs[k + 1]
            acc, src_v, dst_v, rows_v, degt = refs[k + 2:k + 7]
        else:
            part_out = refs[k]
            acc, src_v, dst_v, rows_v = refs[k + 1:k + 5]

        cid = lax.axis_index("c")
        sid = lax.axis_index("s")
        tid = cid * _NS + sid
        row0 = sid * rpt

        zf = jnp.zeros((16,), jnp.float32)
        if with_deg:
            of = jnp.ones((16,), jnp.float32)
            def zdeg(i, _):
                degt[pl.ds(i * 16, 16)] = zf
                return 0
            lax.fori_loop(0, n_acc // 16, zdeg, 0)

        pltpu.sync_copy(src_hbm.at[pl.ds(tid * nb_per_tile, nb_per_tile)], src_v)
        pltpu.sync_copy(dst_hbm.at[pl.ds(tid * nb_per_tile, nb_per_tile)], dst_v)

        for c in range(num_chunks):
            def zrow(i, _):
                for j in range(_LW // 16):
                    rows_v[i, pl.ds(j * 16, 16)] = zf
                return 0
            lax.fori_loop(0, _B, zrow, 0)
            for z in range(rpt // _B):
                pltpu.sync_copy(rows_v, acc.at[pl.ds(row0 + z * _B, _B)])
            plsc.subcore_barrier()

            def ebody(j, _):
                pltpu.sync_copy(tables[c].at[src_v.at[j]], rows_v)
                pltpu.sync_copy(rows_v, acc.at[dst_v.at[j]], add=True)
                if with_deg and c == 0:
                    for l in range(_B // 16):
                        idx = dst_v[j, pl.ds(l * 16, 16)]
                        plsc.addupdate_scatter(degt, [idx], of)
                return 0
            lax.fori_loop(0, nb_per_tile, ebody, 0)

            plsc.subcore_barrier()
            pltpu.sync_copy(acc.at[pl.ds(row0, rpt)],
                            part_out.at[cid, c, pl.ds(row0, rpt)])
            if with_deg and c == 0:
                pltpu.sync_copy(degt, deg_out.at[tid])
            plsc.subcore_barrier()

    return pl.kernel(body, out_type=tuple(out_type), mesh=mesh,
                     scratch_types=scratch,
                     compiler_params=pltpu.CompilerParams(
                         needs_layout_passes=False))


def _dot(a, b):
    return jax.lax.dot_general(a, b, (((1,), (0,)), ((), ())),
                               preferred_element_type=jnp.float32)


def _tc1_body(p_ref, dg_ref, x_ref, w1lT_ref, b1_ref, w1rT_ref, *h_refs):
    p = p_ref[...]
    deg = jnp.sum(dg_ref[...], axis=0)
    inv = 1.0 / jnp.maximum(deg, 1.0)
    w = w1lT_ref[...]
    acc = b1_ref[...]
    for c in range(2):
        mean_c = (p[0, c] + p[1, c]) * inv[:, None]
        acc = acc + _dot(mean_c, w[c * 128:(c + 1) * 128, :])
    acc = acc + _dot(x_ref[...], w1rT_ref[...])
    h = jnp.maximum(acc, 0.0)
    for c in range(len(h_refs)):
        h_refs[c][...] = h[:, c * 128:(c + 1) * 128]


def _tc2_body(p_ref, dg_ref, h0, h1, h2, h3, w2lT_ref, b2_ref, w2rT_ref,
              wcT_ref, bc_ref, out_ref):
    p = p_ref[...]
    deg = jnp.sum(dg_ref[...], axis=0)
    inv = 1.0 / jnp.maximum(deg, 1.0)
    w2l = w2lT_ref[...]
    w2r = w2rT_ref[...]
    acc = b2_ref[...]
    hs = (h0, h1, h2, h3)
    for c in range(4):
        mean_c = (p[0, c] + p[1, c]) * inv[:, None]
        acc = acc + _dot(mean_c, w2l[c * 128:(c + 1) * 128, :])
        acc = acc + _dot(hs[c][...], w2r[c * 128:(c + 1) * 128, :])
    emb = jnp.maximum(acc, 0.0)
    out_ref[...] = _dot(emb, wcT_ref[...]) + bc_ref[...]


def kernel(x, edge_index, W1l, b1, W1r, W2l, b2, W2r, Wc, bc):
    N, D = x.shape
    E = edge_index.shape[1]
    H = W1l.shape[0]
    C = Wc.shape[0]

    ntiles = _NC * _NS
    e_pad = -(-E // (ntiles * _B)) * (ntiles * _B)
    nb_per_tile = e_pad // (ntiles * _B)
    pad = e_pad - E
    src2d = jnp.concatenate(
        [edge_index[0], jnp.zeros((pad,), jnp.int32)]).reshape(-1, _B)
    dst2d = jnp.concatenate(
        [edge_index[1], jnp.full((pad,), N, jnp.int32)]).reshape(-1, _B)

    n_acc = -(-N // (_NS * 32)) * (_NS * 32)
    if n_acc == N:
        n_acc += _NS * 32

    nc1 = D // _LW
    nc2 = H // _LW

    sc1 = _make_sc_agg(nc1, N, n_acc, nb_per_tile, with_deg=True)
    sc2 = _make_sc_agg(nc2, N, n_acc, nb_per_tile, with_deg=False)

    x_chunks = [jax.lax.slice(x, (0, c * _LW), (N, (c + 1) * _LW))
                for c in range(nc1)]
    part1, deg32 = sc1(*x_chunks, src2d, dst2d)

    R = 1024
    grid = (-(-N // R),)
    wspec2 = pl.BlockSpec((D, H), lambda i: (0, 0))
    h_chunks = pl.pallas_call(
        _tc1_body,
        grid=grid,
        in_specs=[
            pl.BlockSpec((_NC, nc1, R, _LW), lambda i: (0, 0, i, 0)),
            pl.BlockSpec((ntiles, R), lambda i: (0, i)),
            pl.BlockSpec((R, D), lambda i: (i, 0)),
            wspec2,
            pl.BlockSpec((1, H), lambda i: (0, 0)),
            wspec2,
        ],
        out_specs=[pl.BlockSpec((R, _LW), lambda i: (i, 0))] * nc2,
        out_shape=[jax.ShapeDtypeStruct((N, _LW), jnp.float32)] * nc2,
    )(part1, deg32, x, W1l.T, b1.reshape(1, H), W1r.T)

    (part2,) = sc2(*h_chunks, src2d, dst2d)

    logits = pl.pallas_call(
        _tc2_body,
        grid=grid,
        in_specs=[
            pl.BlockSpec((_NC, nc2, R, _LW), lambda i: (0, 0, i, 0)),
            pl.BlockSpec((ntiles, R), lambda i: (0, i)),
        ] + [pl.BlockSpec((R, _LW), lambda i: (i, 0))] * nc2 + [
            pl.BlockSpec((H, H), lambda i: (0, 0)),
            pl.BlockSpec((1, H), lambda i: (0, 0)),
            pl.BlockSpec((H, H), lambda i: (0, 0)),
            pl.BlockSpec((H, C), lambda i: (0, 0)),
            pl.BlockSpec((1, C), lambda i: (0, 0)),
        ],
        out_specs=pl.BlockSpec((R, C), lambda i: (i, 0)),
        out_shape=jax.ShapeDtypeStruct((N, C), jnp.float32),
    )(part2, deg32, *h_chunks, W2l.T, b2.reshape(1, H), W2r.T, Wc.T,
      bc.reshape(1, C))

    return logits


# speedup vs baseline: 2.1921x; 1.0339x over previous
import jax
import jax.numpy as jnp
from jax import lax
from jax.experimental import pallas as pl
from jax.experimental.pallas import tpu as pltpu
from jax.experimental.pallas import tpu_sc as plsc

_NC = 2
_NS = 16
_B = 64
_LW = 128


def _make_sc_agg(num_chunks: int, n_nodes: int, n_acc: int, nb_per_tile: int,
                 with_deg: bool):
    rpt = n_acc // _NS
    assert rpt % _B == 0

    mesh = plsc.VectorSubcoreMesh(core_axis_name="c", subcore_axis_name="s")

    out_type = [jax.ShapeDtypeStruct((_NC, num_chunks, n_acc, _LW), jnp.float32)]
    if with_deg:
        out_type.append(jax.ShapeDtypeStruct((_NC * _NS, n_acc), jnp.float32))

    scratch = [
        pltpu.VMEM_SHARED((n_acc, _LW), jnp.float32),
        pltpu.VMEM((nb_per_tile, _B), jnp.int32),
        pltpu.VMEM((nb_per_tile, _B), jnp.int32),
        pltpu.VMEM((_B, _LW), jnp.float32),
        pltpu.VMEM((_B, _LW), jnp.float32),
        pltpu.SemaphoreType.DMA,
        pltpu.SemaphoreType.DMA,
        pltpu.SemaphoreType.DMA,
        pltpu.SemaphoreType.DMA,
    ]
    if with_deg:
        scratch.append(pltpu.VMEM((n_acc,), jnp.float32))

    def body(*refs):
        tables = refs[:num_chunks]
        src_hbm, dst_hbm = refs[num_chunks], refs[num_chunks + 1]
        k = num_chunks + 2
        if with_deg:
            part_out, deg_out = refs[k], refs[k + 1]
            k += 2
        else:
            part_out = refs[k]
            k += 1
        acc, src_v, dst_v = refs[k:k + 3]
        bufs = refs[k + 3:k + 5]
        gsems = refs[k + 5:k + 7]
        ssems = refs[k + 7:k + 9]
        if with_deg:
            degt = refs[k + 9]

        cid = lax.axis_index("c")
        sid = lax.axis_index("s")
        tid = cid * _NS + sid
        row0 = sid * rpt

        zf = jnp.zeros((16,), jnp.float32)
        if with_deg:
            of = jnp.ones((16,), jnp.float32)
            def zdeg(i, _):
                degt[pl.ds(i * 16, 16)] = zf
                return 0
            lax.fori_loop(0, n_acc // 16, zdeg, 0)

        pltpu.sync_copy(src_hbm.at[pl.ds(tid * nb_per_tile, nb_per_tile)], src_v)
        pltpu.sync_copy(dst_hbm.at[pl.ds(tid * nb_per_tile, nb_per_tile)], dst_v)

        nb2 = nb_per_tile // 2
        for c in range(num_chunks):
            def zrow(i, _):
                for j in range(_LW // 16):
                    bufs[0][i, pl.ds(j * 16, 16)] = zf
                return 0
            lax.fori_loop(0, _B, zrow, 0)
            for z in range(rpt // _B):
                pltpu.sync_copy(bufs[0], acc.at[pl.ds(row0 + z * _B, _B)])
            plsc.subcore_barrier()

            def pbody(p, _):
                for b in range(2):
                    j = 2 * p + b

                    @pl.when(p > 0)
                    def _wait_prev_scatter():
                        pltpu.make_async_copy(
                            bufs[b], acc.at[dst_v.at[j - 2]], ssems[b]).wait()

                    gd = pltpu.async_copy(tables[c].at[src_v.at[j]],
                                          bufs[b], gsems[b])
                    if with_deg and c == 0:
                        for l in range(_B // 16):
                            idx = dst_v[j, pl.ds(l * 16, 16)]
                            plsc.addupdate_scatter(degt, [idx], of)
                    gd.wait()
                    pltpu.async_copy(bufs[b], acc.at[dst_v.at[j]],
                                     ssems[b], add=True)
                return 0
            lax.fori_loop(0, nb2, pbody, 0)
            for b in range(2):
                pltpu.make_async_copy(
                    bufs[b], acc.at[dst_v.at[nb_per_tile - 2 + b]],
                    ssems[b]).wait()

            plsc.subcore_barrier()
            pltpu.sync_copy(acc.at[pl.ds(row0, rpt)],
                            part_out.at[cid, c, pl.ds(row0, rpt)])
            if with_deg and c == 0:
                pltpu.sync_copy(degt, deg_out.at[tid])
            plsc.subcore_barrier()

    return pl.kernel(body, out_type=tuple(out_type), mesh=mesh,
                     scratch_types=scratch,
                     compiler_params=pltpu.CompilerParams(
                         needs_layout_passes=False))


def _dot(a, b):
    return jax.lax.dot_general(a, b, (((1,), (0,)), ((), ())),
                               preferred_element_type=jnp.float32)


def _tc1_body(p_ref, dg_ref, x_ref, w1lT_ref, b1_ref, w1rT_ref, *h_refs):
    p = p_ref[...]
    deg = jnp.sum(dg_ref[...], axis=0)
    inv = 1.0 / jnp.maximum(deg, 1.0)
    w = w1lT_ref[...]
    acc = b1_ref[...]
    for c in range(2):
        mean_c = (p[0, c] + p[1, c]) * inv[:, None]
        acc = acc + _dot(mean_c, w[c * 128:(c + 1) * 128, :])
    acc = acc + _dot(x_ref[...], w1rT_ref[...])
    h = jnp.maximum(acc, 0.0)
    for c in range(len(h_refs)):
        h_refs[c][...] = h[:, c * 128:(c + 1) * 128]


def _tc2_body(p_ref, dg_ref, h0, h1, h2, h3, w2lT_ref, b2_ref, w2rT_ref,
              wcT_ref, bc_ref, out_ref):
    p = p_ref[...]
    deg = jnp.sum(dg_ref[...], axis=0)
    inv = 1.0 / jnp.maximum(deg, 1.0)
    w2l = w2lT_ref[...]
    w2r = w2rT_ref[...]
    acc = b2_ref[...]
    hs = (h0, h1, h2, h3)
    for c in range(4):
        mean_c = (p[0, c] + p[1, c]) * inv[:, None]
        acc = acc + _dot(mean_c, w2l[c * 128:(c + 1) * 128, :])
        acc = acc + _dot(hs[c][...], w2r[c * 128:(c + 1) * 128, :])
    emb = jnp.maximum(acc, 0.0)
    out_ref[...] = _dot(emb, wcT_ref[...]) + bc_ref[...]


def kernel(x, edge_index, W1l, b1, W1r, W2l, b2, W2r, Wc, bc):
    N, D = x.shape
    E = edge_index.shape[1]
    H = W1l.shape[0]
    C = Wc.shape[0]

    ntiles = _NC * _NS
    e_pad = -(-E // (ntiles * _B * 2)) * (ntiles * _B * 2)
    nb_per_tile = e_pad // (ntiles * _B)
    pad = e_pad - E
    src2d = jnp.concatenate(
        [edge_index[0], jnp.zeros((pad,), jnp.int32)]).reshape(-1, _B)
    dst2d = jnp.concatenate(
        [edge_index[1], jnp.full((pad,), N, jnp.int32)]).reshape(-1, _B)

    n_acc = -(-N // (_NS * 32)) * (_NS * 32)
    if n_acc == N:
        n_acc += _NS * 32

    nc1 = D // _LW
    nc2 = H // _LW

    sc1 = _make_sc_agg(nc1, N, n_acc, nb_per_tile, with_deg=True)
    sc2 = _make_sc_agg(nc2, N, n_acc, nb_per_tile, with_deg=False)

    x_chunks = [jax.lax.slice(x, (0, c * _LW), (N, (c + 1) * _LW))
                for c in range(nc1)]
    part1, deg32 = sc1(*x_chunks, src2d, dst2d)

    R = 1024
    grid = (-(-N // R),)
    wspec2 = pl.BlockSpec((D, H), lambda i: (0, 0))
    h_chunks = pl.pallas_call(
        _tc1_body,
        grid=grid,
        in_specs=[
            pl.BlockSpec((_NC, nc1, R, _LW), lambda i: (0, 0, i, 0)),
            pl.BlockSpec((ntiles, R), lambda i: (0, i)),
            pl.BlockSpec((R, D), lambda i: (i, 0)),
            wspec2,
            pl.BlockSpec((1, H), lambda i: (0, 0)),
            wspec2,
        ],
        out_specs=[pl.BlockSpec((R, _LW), lambda i: (i, 0))] * nc2,
        out_shape=[jax.ShapeDtypeStruct((N, _LW), jnp.float32)] * nc2,
    )(part1, deg32, x, W1l.T, b1.reshape(1, H), W1r.T)

    (part2,) = sc2(*h_chunks, src2d, dst2d)

    logits = pl.pallas_call(
        _tc2_body,
        grid=grid,
        in_specs=[
            pl.BlockSpec((_NC, nc2, R, _LW), lambda i: (0, 0, i, 0)),
            pl.BlockSpec((ntiles, R), lambda i: (0, i)),
        ] + [pl.BlockSpec((R, _LW), lambda i: (i, 0))] * nc2 + [
            pl.BlockSpec((H, H), lambda i: (0, 0)),
            pl.BlockSpec((1, H), lambda i: (0, 0)),
            pl.BlockSpec((H, H), lambda i: (0, 0)),
            pl.BlockSpec((H, C), lambda i: (0, 0)),
            pl.BlockSpec((1, C), lambda i: (0, 0)),
        ],
        out_specs=pl.BlockSpec((R, C), lambda i: (i, 0)),
        out_shape=jax.ShapeDtypeStruct((N, C), jnp.float32),
    )(part2, deg32, *h_chunks, W2l.T, b2.reshape(1, H), W2r.T, Wc.T,
      bc.reshape(1, C))

    return logits
